# baseline (speedup 1.0000x reference)
import math

import jax
import jax.numpy as jnp
from jax.experimental import pallas as pl
from jax.experimental.pallas import tpu as pltpu
from jax.experimental.pallas import tpu_sc as plsc

EMB = 128
WINDOW = 256
SCALE = math.sqrt(EMB)
ROWS_PER_BLOCK = 10000


def _scale_table(table):
    def body(x_ref, o_ref):
        o_ref[...] = x_ref[...] * SCALE

    return pl.pallas_call(
        body,
        out_shape=jax.ShapeDtypeStruct(table.shape, table.dtype),
        grid=(table.shape[0] // ROWS_PER_BLOCK,),
        in_specs=[pl.BlockSpec((ROWS_PER_BLOCK, EMB), lambda i: (i, 0))],
        out_specs=pl.BlockSpec((ROWS_PER_BLOCK, EMB), lambda i: (i, 0)),
    )(table)


def _gather(table, idx3):
    b = idx3.shape[0]
    l = idx3.shape[2]
    mesh = plsc.VectorSubcoreMesh(core_axis_name="core", subcore_axis_name="subcore")

    bb = 8
    e = table.shape[1]

    @pl.kernel(
        out_type=jax.ShapeDtypeStruct((b, l, e), table.dtype),
        mesh=mesh,
        scratch_types=[pltpu.SemaphoreType.DMA],
    )
    def kern(x_hbm, i_hbm, o_hbm, sem):
        def body(i_vmem, o_vmem):
            copies = [
                pltpu.async_copy(x_hbm.at[i_vmem.at[j, 0]], o_vmem.at[j], sem)
                for j in range(bb)
            ]
            for c in copies:
                c.wait()

        pltpu.emit_pipeline(
            body,
            grid=(b // bb,),
            in_specs=[pl.BlockSpec((bb, 1, l), index_map=lambda i: (i, 0, 0))],
            out_specs=[pl.BlockSpec((bb, l, e), index_map=lambda i: (i, 0, 0))],
            core_axis_name=("core", "subcore"),
            dimension_semantics=(pltpu.PARALLEL,),
        )(i_hbm, o_hbm)

    return kern(table, idx3)


def kernel(table, y):
    b, l = y.shape
    idx = y.reshape(b, 1, l).astype(jnp.int32)
    st = _scale_table(table)
    h = b // 2
    g1 = _gather(st, idx[:h])
    g2 = _gather(st, idx[h:])
    out = jnp.zeros((b, l, EMB), dtype=table.dtype)
    out = jax.lax.dynamic_update_slice(out, g1, (0, 0, 0))
    out = jax.lax.dynamic_update_slice(out, g2, (h, 0, 0))
    return out


# speedup vs baseline: 1.5639x; 1.5639x over previous
import math

import jax
import jax.numpy as jnp
from jax.experimental import pallas as pl
from jax.experimental.pallas import tpu as pltpu
from jax.experimental.pallas import tpu_sc as plsc

EMB = 128
WINDOW = 256
SCALE = math.sqrt(EMB)
ROWS_PER_BLOCK = 10000


def _scale_table(table):
    def body(x_ref, o_ref):
        o_ref[...] = x_ref[...] * SCALE

    return pl.pallas_call(
        body,
        out_shape=jax.ShapeDtypeStruct(table.shape, table.dtype),
        grid=(table.shape[0] // ROWS_PER_BLOCK,),
        in_specs=[pl.BlockSpec((ROWS_PER_BLOCK, EMB), lambda i: (i, 0))],
        out_specs=pl.BlockSpec((ROWS_PER_BLOCK, EMB), lambda i: (i, 0)),
    )(table)


def _gather(table, idx3):
    b = idx3.shape[0]
    l = idx3.shape[2]
    mesh = plsc.VectorSubcoreMesh(core_axis_name="core", subcore_axis_name="subcore")

    bb = 8
    e = table.shape[1]

    @pl.kernel(
        out_type=jax.ShapeDtypeStruct((b, l, e), table.dtype),
        mesh=mesh,
        scratch_types=[pltpu.SemaphoreType.DMA],
    )
    def kern(x_hbm, i_hbm, o_hbm, sem):
        def body(i_vmem, o_vmem):
            copies = [
                pltpu.async_copy(x_hbm.at[i_vmem.at[j, 0]], o_vmem.at[j], sem)
                for j in range(bb)
            ]
            for c in copies:
                c.wait()

        pltpu.emit_pipeline(
            body,
            grid=(b // bb,),
            in_specs=[pl.BlockSpec((bb, 1, l), index_map=lambda i: (i, 0, 0))],
            out_specs=[pl.BlockSpec((bb, l, e), index_map=lambda i: (i, 0, 0))],
            core_axis_name=("core", "subcore"),
            dimension_semantics=(pltpu.PARALLEL,),
        )(i_hbm, o_hbm)

    return kern(table, idx3)


def kernel(table, y):
    b, l = y.shape
    idx = y.reshape(b, 1, l).astype(jnp.int32)
    return _gather(_scale_table(table), idx)


# speedup vs baseline: 1.5773x; 1.0086x over previous
import math

import jax
import jax.numpy as jnp
from jax.experimental import pallas as pl
from jax.experimental.pallas import tpu as pltpu
from jax.experimental.pallas import tpu_sc as plsc

EMB = 128
WINDOW = 256
SCALE = math.sqrt(EMB)
ROWS_PER_BLOCK = 20000


def _scale_table(table):
    def body(x_ref, o_ref):
        o_ref[...] = x_ref[...] * SCALE

    return pl.pallas_call(
        body,
        out_shape=jax.ShapeDtypeStruct(table.shape, table.dtype),
        grid=(table.shape[0] // ROWS_PER_BLOCK,),
        in_specs=[pl.BlockSpec((ROWS_PER_BLOCK, EMB), lambda i: (i, 0))],
        out_specs=pl.BlockSpec((ROWS_PER_BLOCK, EMB), lambda i: (i, 0)),
    )(table)


def _gather(table, idx3):
    b = idx3.shape[0]
    l = idx3.shape[2]
    mesh = plsc.VectorSubcoreMesh(core_axis_name="core", subcore_axis_name="subcore")

    bb = 8
    e = table.shape[1]

    @pl.kernel(
        out_type=jax.ShapeDtypeStruct((b, l, e), table.dtype),
        mesh=mesh,
        scratch_types=[pltpu.SemaphoreType.DMA],
    )
    def kern(x_hbm, i_hbm, o_hbm, sem):
        def body(i_vmem, o_vmem):
            copies = [
                pltpu.async_copy(x_hbm.at[i_vmem.at[j, 0]], o_vmem.at[j], sem)
                for j in range(bb)
            ]
            for c in copies:
                c.wait()

        pltpu.emit_pipeline(
            body,
            grid=(b // bb,),
            in_specs=[pl.BlockSpec((bb, 1, l), index_map=lambda i: (i, 0, 0))],
            out_specs=[pl.BlockSpec((bb, l, e), index_map=lambda i: (i, 0, 0))],
            core_axis_name=("core", "subcore"),
            dimension_semantics=(pltpu.PARALLEL,),
        )(i_hbm, o_hbm)

    return kern(table, idx3)


def kernel(table, y):
    b, l = y.shape
    idx = y.reshape(b, 1, l).astype(jnp.int32)
    return _gather(_scale_table(table), idx)


# speedup vs baseline: 1.5784x; 1.0007x over previous
import math

import jax
import jax.numpy as jnp
from jax.experimental import pallas as pl
from jax.experimental.pallas import tpu as pltpu
from jax.experimental.pallas import tpu_sc as plsc

EMB = 128
WINDOW = 256
SCALE = math.sqrt(EMB)
ROWS_PER_BLOCK = 25000


def _scale_table(table):
    def body(x_ref, o_ref):
        o_ref[...] = x_ref[...] * SCALE

    return pl.pallas_call(
        body,
        out_shape=jax.ShapeDtypeStruct(table.shape, table.dtype),
        grid=(table.shape[0] // ROWS_PER_BLOCK,),
        in_specs=[pl.BlockSpec((ROWS_PER_BLOCK, EMB), lambda i: (i, 0))],
        out_specs=pl.BlockSpec((ROWS_PER_BLOCK, EMB), lambda i: (i, 0)),
    )(table)


def _gather(table, idx3):
    b = idx3.shape[0]
    l = idx3.shape[2]
    mesh = plsc.VectorSubcoreMesh(core_axis_name="core", subcore_axis_name="subcore")

    bb = 8
    e = table.shape[1]

    @pl.kernel(
        out_type=jax.ShapeDtypeStruct((b, l, e), table.dtype),
        mesh=mesh,
        scratch_types=[pltpu.SemaphoreType.DMA],
    )
    def kern(x_hbm, i_hbm, o_hbm, sem):
        def body(i_vmem, o_vmem):
            copies = [
                pltpu.async_copy(x_hbm.at[i_vmem.at[j, 0]], o_vmem.at[j], sem)
                for j in range(bb)
            ]
            for c in copies:
                c.wait()

        pltpu.emit_pipeline(
            body,
            grid=(b // bb,),
            in_specs=[pl.BlockSpec((bb, 1, l), index_map=lambda i: (i, 0, 0))],
            out_specs=[pl.BlockSpec((bb, l, e), index_map=lambda i: (i, 0, 0))],
            core_axis_name=("core", "subcore"),
            dimension_semantics=(pltpu.PARALLEL,),
        )(i_hbm, o_hbm)

    return kern(table, idx3)


def kernel(table, y):
    b, l = y.shape
    idx = y.reshape(b, 1, l).astype(jnp.int32)
    return _gather(_scale_table(table), idx)
